# baseline (speedup 1.0000x reference)
import jax
import jax.numpy as jnp
from jax import lax
from jax.experimental import pallas as pl
from jax.experimental.pallas import tpu as pltpu
from jax.experimental.pallas import tpu_sc as plsc

N = 10000
E = 320000
D = 128
NC, NS = 2, 16
NW = NC * NS
EP = 327680
EPT = EP // NW
K = 64
NBLK = EPT // K
NRB = 4
NIB = 8
ACC_R = 10240
ZRPT = ACC_R // NS


def _seg_kernel(with_deg: bool):
    mesh = plsc.VectorSubcoreMesh(core_axis_name="c", subcore_axis_name="s",
                                  num_cores=NC, num_subcores=NS)
    out_type = [jax.ShapeDtypeStruct((NC, ACC_R, D), jnp.float32)]
    scratch = (
        [pltpu.VMEM((2, K), jnp.int32) for _ in range(NIB)]
        + [pltpu.VMEM((K, D), jnp.float32) for _ in range(NRB)]
        + [pltpu.VMEM((16, D), jnp.float32),
           pltpu.VMEM_SHARED((ACC_R, D), jnp.float32)]
        + [pltpu.SemaphoreType.DMA for _ in range(NIB)]
        + [pltpu.SemaphoreType.DMA for _ in range(NRB)]
        + [pltpu.SemaphoreType.DMA]
    )
    if with_deg:
        out_type.append(jax.ShapeDtypeStruct((NC, ACC_R), jnp.float32))
        scratch += [
            pltpu.VMEM((K,), jnp.float32),
            pltpu.VMEM((K,), jnp.float32),
            pltpu.VMEM_SHARED((ACC_R,), jnp.float32),
            pltpu.SemaphoreType.DMA,
        ]

    def body(feat_hbm, idx_hbm, *rest):
        no = 2 if with_deg else 1
        outs, rest = rest[:no], rest[no:]
        if with_deg:
            acc_out, deg_out = outs
            rest, (ones1, zde, deg_sh, semd) = rest[:-4], rest[-4:]
        else:
            acc_out, = outs
        idxb = rest[:NIB]
        rows = rest[NIB:NIB + NRB]
        zbuf, acc_sh = rest[NIB + NRB:NIB + NRB + 2]
        semi = rest[NIB + NRB + 2:NIB + NRB + 2 + NIB]
        semr = rest[NIB + NRB + 2 + NIB:NIB + NRB + 2 + NIB + NRB]
        sems = rest[NIB + NRB + 2 + NIB + NRB]
        cid = lax.axis_index("c")
        sid = lax.axis_index("s")
        wid = cid * NS + sid
        blk0 = wid * NBLK

        def fill(i, _):
            r = i // 8
            c = (i % 8) * 16
            zbuf[r, pl.ds(c, 16)] = jnp.zeros((16,), jnp.float32)
            if with_deg:
                o = (i % (K // 16)) * 16
                ones1[pl.ds(o, 16)] = jnp.ones((16,), jnp.float32)
                zde[pl.ds(o, 16)] = jnp.zeros((16,), jnp.float32)
            return 0
        lax.fori_loop(0, 16 * 8, fill, 0)

        def zero(b, _):
            pltpu.sync_copy(zbuf, acc_sh.at[pl.ds(sid * ZRPT + b * 16, 16)])
            return 0
        lax.fori_loop(0, ZRPT // 16, zero, 0)
        if with_deg:
            def zero2(b, _):
                pltpu.sync_copy(zde, deg_sh.at[pl.ds(sid * ZRPT + b * K, K)])
                return 0
            lax.fori_loop(0, ZRPT // K, zero2, 0)
        plsc.subcore_barrier()

        for q in range(NIB):
            pltpu.async_copy(idx_hbm.at[blk0 + q], idxb[q], semi[q])
        for j in range(NRB):
            pltpu.make_async_copy(idx_hbm.at[blk0 + j], idxb[j],
                                  semi[j]).wait()
            pltpu.async_copy(feat_hbm.at[idxb[j].at[0]], rows[j], semr[j])

        def octet(p, _):
            for j in range(NIB):
                b = NIB * p + j
                rj = j % NRB
                pltpu.make_async_copy(feat_hbm.at[idxb[j].at[0]],
                                      rows[rj], semr[rj]).wait()
                cp = pltpu.async_copy(rows[rj], acc_sh.at[idxb[j].at[1]],
                                      sems, add=True)
                if with_deg:
                    cpd = pltpu.async_copy(ones1, deg_sh.at[idxb[j].at[1]],
                                           semd, add=True)
                cp.wait()
                if with_deg:
                    cpd.wait()

                @pl.when(b + NIB < NBLK)
                def _():
                    pltpu.async_copy(idx_hbm.at[blk0 + b + NIB], idxb[j],
                                     semi[j])

                @pl.when(b + NRB < NBLK)
                def _():
                    qn = (j + NRB) % NIB
                    pltpu.make_async_copy(idx_hbm.at[blk0 + b + NRB],
                                          idxb[qn], semi[qn]).wait()
                    pltpu.async_copy(feat_hbm.at[idxb[qn].at[0]],
                                     rows[rj], semr[rj])
            return 0
        lax.fori_loop(0, NBLK // NIB, octet, 0)
        plsc.subcore_barrier()

        pltpu.sync_copy(acc_sh.at[pl.ds(sid * ZRPT, ZRPT)],
                        acc_out.at[cid, pl.ds(sid * ZRPT, ZRPT)])
        if with_deg:
            pltpu.sync_copy(deg_sh.at[pl.ds(sid * ZRPT, ZRPT)],
                            deg_out.at[cid, pl.ds(sid * ZRPT, ZRPT)])

    return pl.kernel(body, out_type=out_type, mesh=mesh,
                     scratch_types=scratch)


BR = 1024


def _dense_body(p0, p1, g, wt1, bt1, wt2, bt2, wp1, bp1, wp2, bp2, dout):
    rec = 1.0 / jnp.maximum(g[:, :], 1.0)
    a = (p0[:, :] + p1[:, :]) * rec
    hi = jax.lax.Precision.HIGHEST
    ht = jnp.maximum(
        jnp.dot(a, wt1[:, :], precision=hi,
                preferred_element_type=jnp.float32) + bt1[:, :], 0.0)
    hp = jnp.maximum(
        jnp.dot(a, wp1[:, :], precision=hi,
                preferred_element_type=jnp.float32) + bp1[:, :], 0.0)
    dout[:, :] = (jnp.dot(ht, wt2[:, :], precision=hi,
                          preferred_element_type=jnp.float32)
                  - jnp.dot(hp, wp2[:, :], precision=hi,
                            preferred_element_type=jnp.float32))


def _loss_body(p0, p1, g, db, out):
    i = pl.program_id(0)
    rec = 1.0 / jnp.maximum(g[:, :], 1.0)
    t = (p0[:, :] + p1[:, :]) * rec + db[:, :]
    row = i * BR + jax.lax.broadcasted_iota(jnp.int32, (BR, D), 0)
    t = jnp.where(row < N, t, 0.0)
    s = jnp.reshape(jnp.sum(t * t) * (1.0 / (N * D)), (1, 1))

    @pl.when(i == 0)
    def _():
        out[:, :] = s

    @pl.when(i > 0)
    def _():
        out[:, :] += s


def _row_spec(w):
    return pl.BlockSpec((BR, w), lambda i: (i, 0))


def _full_spec(r, c):
    return pl.BlockSpec((r, c), lambda i: (0, 0))


@jax.jit
def kernel(x, edge_index, Wt1, bt1, Wt2, bt2, Wp1, bp1, Wp2, bp2):
    src = edge_index[0].astype(jnp.int32)
    dst = edge_index[1].astype(jnp.int32)
    pad = EP - E
    src_p = jnp.concatenate([src, jnp.zeros((pad,), jnp.int32)])
    dst_p = jnp.concatenate([dst, jnp.full((pad,), N, jnp.int32)])
    idx3 = jnp.stack([src_p.reshape(EP // K, K),
                      dst_p.reshape(EP // K, K)], axis=1)

    acc1, degp = _seg_kernel(with_deg=True)(x, idx3)
    deg = (degp[0] + degp[1]).reshape(ACC_R, 1)

    d = pl.pallas_call(
        _dense_body,
        grid=(ACC_R // BR,),
        in_specs=[_row_spec(D), _row_spec(D), _row_spec(1),
                  _full_spec(D, D), _full_spec(1, D), _full_spec(D, D),
                  _full_spec(1, D), _full_spec(D, D), _full_spec(1, D),
                  _full_spec(D, D), _full_spec(1, D)],
        out_specs=_row_spec(D),
        out_shape=jax.ShapeDtypeStruct((ACC_R, D), jnp.float32),
    )(acc1[0], acc1[1], deg,
      Wt1, bt1.reshape(1, D), Wt2, bt2.reshape(1, D),
      Wp1, bp1.reshape(1, D), Wp2, bp2.reshape(1, D))

    acc2, = _seg_kernel(with_deg=False)(d, idx3)

    db = (bt2 - bp2).reshape(1, D)
    loss = pl.pallas_call(
        _loss_body,
        grid=(ACC_R // BR,),
        in_specs=[_row_spec(D), _row_spec(D), _row_spec(1), _full_spec(1, D)],
        out_specs=_full_spec(1, 1),
        out_shape=jax.ShapeDtypeStruct((1, 1), jnp.float32),
    )(acc2[0], acc2[1], deg, db)

    return loss[0, 0]


# speedup vs baseline: 1.0520x; 1.0520x over previous
import jax
import jax.numpy as jnp
from jax import lax
from jax.experimental import pallas as pl
from jax.experimental.pallas import tpu as pltpu
from jax.experimental.pallas import tpu_sc as plsc

N = 10000
E = 320000
D = 128
NC, NS = 2, 16
NW = NC * NS
EP = 327680
EPT = EP // NW
K = 128
NBLK = EPT // K
NRB = 2
NIB = 4
ACC_R = 10240
ZRPT = ACC_R // NS


def _seg_kernel(with_deg: bool):
    mesh = plsc.VectorSubcoreMesh(core_axis_name="c", subcore_axis_name="s",
                                  num_cores=NC, num_subcores=NS)
    out_type = [jax.ShapeDtypeStruct((NC, ACC_R, D), jnp.float32)]
    scratch = (
        [pltpu.VMEM((2, K), jnp.int32) for _ in range(NIB)]
        + [pltpu.VMEM((K, D), jnp.float32) for _ in range(NRB)]
        + [pltpu.VMEM((16, D), jnp.float32),
           pltpu.VMEM_SHARED((ACC_R, D), jnp.float32)]
        + [pltpu.SemaphoreType.DMA for _ in range(NIB)]
        + [pltpu.SemaphoreType.DMA for _ in range(NRB)]
        + [pltpu.SemaphoreType.DMA]
    )
    if with_deg:
        out_type.append(jax.ShapeDtypeStruct((NC, ACC_R), jnp.float32))
        scratch += [
            pltpu.VMEM((K,), jnp.float32),
            pltpu.VMEM((K,), jnp.float32),
            pltpu.VMEM_SHARED((ACC_R,), jnp.float32),
            pltpu.SemaphoreType.DMA,
        ]

    def body(feat_hbm, idx_hbm, *rest):
        no = 2 if with_deg else 1
        outs, rest = rest[:no], rest[no:]
        if with_deg:
            acc_out, deg_out = outs
            rest, (ones1, zde, deg_sh, semd) = rest[:-4], rest[-4:]
        else:
            acc_out, = outs
        idxb = rest[:NIB]
        rows = rest[NIB:NIB + NRB]
        zbuf, acc_sh = rest[NIB + NRB:NIB + NRB + 2]
        semi = rest[NIB + NRB + 2:NIB + NRB + 2 + NIB]
        semr = rest[NIB + NRB + 2 + NIB:NIB + NRB + 2 + NIB + NRB]
        sems = rest[NIB + NRB + 2 + NIB + NRB]
        cid = lax.axis_index("c")
        sid = lax.axis_index("s")
        wid = cid * NS + sid
        blk0 = wid * NBLK

        def fill(i, _):
            r = i // 8
            c = (i % 8) * 16
            zbuf[r, pl.ds(c, 16)] = jnp.zeros((16,), jnp.float32)
            if with_deg:
                o = (i % (K // 16)) * 16
                ones1[pl.ds(o, 16)] = jnp.ones((16,), jnp.float32)
                zde[pl.ds(o, 16)] = jnp.zeros((16,), jnp.float32)
            return 0
        lax.fori_loop(0, 16 * 8, fill, 0)

        def zero(b, _):
            pltpu.sync_copy(zbuf, acc_sh.at[pl.ds(sid * ZRPT + b * 16, 16)])
            return 0
        lax.fori_loop(0, ZRPT // 16, zero, 0)
        if with_deg:
            def zero2(b, _):
                pltpu.sync_copy(zde, deg_sh.at[pl.ds(sid * ZRPT + b * K, K)])
                return 0
            lax.fori_loop(0, ZRPT // K, zero2, 0)
        plsc.subcore_barrier()

        for q in range(NIB):
            pltpu.async_copy(idx_hbm.at[blk0 + q], idxb[q], semi[q])
        for j in range(NRB):
            pltpu.make_async_copy(idx_hbm.at[blk0 + j], idxb[j],
                                  semi[j]).wait()
            pltpu.async_copy(feat_hbm.at[idxb[j].at[0]], rows[j], semr[j])

        def octet(p, _):
            for j in range(NIB):
                b = NIB * p + j
                rj = j % NRB
                pltpu.make_async_copy(feat_hbm.at[idxb[j].at[0]],
                                      rows[rj], semr[rj]).wait()
                cp = pltpu.async_copy(rows[rj], acc_sh.at[idxb[j].at[1]],
                                      sems, add=True)
                if with_deg:
                    cpd = pltpu.async_copy(ones1, deg_sh.at[idxb[j].at[1]],
                                           semd, add=True)
                cp.wait()
                if with_deg:
                    cpd.wait()

                @pl.when(b + NIB < NBLK)
                def _():
                    pltpu.async_copy(idx_hbm.at[blk0 + b + NIB], idxb[j],
                                     semi[j])

                @pl.when(b + NRB < NBLK)
                def _():
                    qn = (j + NRB) % NIB
                    pltpu.make_async_copy(idx_hbm.at[blk0 + b + NRB],
                                          idxb[qn], semi[qn]).wait()
                    pltpu.async_copy(feat_hbm.at[idxb[qn].at[0]],
                                     rows[rj], semr[rj])
            return 0
        lax.fori_loop(0, NBLK // NIB, octet, 0)
        plsc.subcore_barrier()

        pltpu.sync_copy(acc_sh.at[pl.ds(sid * ZRPT, ZRPT)],
                        acc_out.at[cid, pl.ds(sid * ZRPT, ZRPT)])
        if with_deg:
            pltpu.sync_copy(deg_sh.at[pl.ds(sid * ZRPT, ZRPT)],
                            deg_out.at[cid, pl.ds(sid * ZRPT, ZRPT)])

    return pl.kernel(body, out_type=out_type, mesh=mesh,
                     scratch_types=scratch)


BR = 1024


def _dense_body(p0, p1, g, wt1, bt1, wt2, bt2, wp1, bp1, wp2, bp2, dout):
    rec = 1.0 / jnp.maximum(g[:, :], 1.0)
    a = (p0[:, :] + p1[:, :]) * rec
    hi = jax.lax.Precision.HIGHEST
    ht = jnp.maximum(
        jnp.dot(a, wt1[:, :], precision=hi,
                preferred_element_type=jnp.float32) + bt1[:, :], 0.0)
    hp = jnp.maximum(
        jnp.dot(a, wp1[:, :], precision=hi,
                preferred_element_type=jnp.float32) + bp1[:, :], 0.0)
    dout[:, :] = (jnp.dot(ht, wt2[:, :], precision=hi,
                          preferred_element_type=jnp.float32)
                  - jnp.dot(hp, wp2[:, :], precision=hi,
                            preferred_element_type=jnp.float32))


def _loss_body(p0, p1, g, db, out):
    i = pl.program_id(0)
    rec = 1.0 / jnp.maximum(g[:, :], 1.0)
    t = (p0[:, :] + p1[:, :]) * rec + db[:, :]
    row = i * BR + jax.lax.broadcasted_iota(jnp.int32, (BR, D), 0)
    t = jnp.where(row < N, t, 0.0)
    s = jnp.reshape(jnp.sum(t * t) * (1.0 / (N * D)), (1, 1))

    @pl.when(i == 0)
    def _():
        out[:, :] = s

    @pl.when(i > 0)
    def _():
        out[:, :] += s


def _row_spec(w):
    return pl.BlockSpec((BR, w), lambda i: (i, 0))


def _full_spec(r, c):
    return pl.BlockSpec((r, c), lambda i: (0, 0))


@jax.jit
def kernel(x, edge_index, Wt1, bt1, Wt2, bt2, Wp1, bp1, Wp2, bp2):
    src = edge_index[0].astype(jnp.int32)
    dst = edge_index[1].astype(jnp.int32)
    pad = EP - E
    src_p = jnp.concatenate([src, jnp.zeros((pad,), jnp.int32)])
    dst_p = jnp.concatenate([dst, jnp.full((pad,), N, jnp.int32)])
    idx3 = jnp.stack([src_p.reshape(EP // K, K),
                      dst_p.reshape(EP // K, K)], axis=1)

    acc1, degp = _seg_kernel(with_deg=True)(x, idx3)
    deg = (degp[0] + degp[1]).reshape(ACC_R, 1)

    d = pl.pallas_call(
        _dense_body,
        grid=(ACC_R // BR,),
        in_specs=[_row_spec(D), _row_spec(D), _row_spec(1),
                  _full_spec(D, D), _full_spec(1, D), _full_spec(D, D),
                  _full_spec(1, D), _full_spec(D, D), _full_spec(1, D),
                  _full_spec(D, D), _full_spec(1, D)],
        out_specs=_row_spec(D),
        out_shape=jax.ShapeDtypeStruct((ACC_R, D), jnp.float32),
    )(acc1[0], acc1[1], deg,
      Wt1, bt1.reshape(1, D), Wt2, bt2.reshape(1, D),
      Wp1, bp1.reshape(1, D), Wp2, bp2.reshape(1, D))

    acc2, = _seg_kernel(with_deg=False)(d, idx3)

    db = (bt2 - bp2).reshape(1, D)
    loss = pl.pallas_call(
        _loss_body,
        grid=(ACC_R // BR,),
        in_specs=[_row_spec(D), _row_spec(D), _row_spec(1), _full_spec(1, D)],
        out_specs=_full_spec(1, 1),
        out_shape=jax.ShapeDtypeStruct((1, 1), jnp.float32),
    )(acc2[0], acc2[1], deg, db)

    return loss[0, 0]


# speedup vs baseline: 1.0547x; 1.0026x over previous
import jax
import jax.numpy as jnp
from jax import lax
from jax.experimental import pallas as pl
from jax.experimental.pallas import tpu as pltpu
from jax.experimental.pallas import tpu_sc as plsc

N = 10000
E = 320000
D = 128
NC, NS = 2, 16
NW = NC * NS
EP = 327680
EPT = EP // NW
K = 128
NBLK = EPT // K
NRB = 2
NIB = 4
ACC_R = 10240
ZRPT = ACC_R // NS


def _seg_kernel(with_deg: bool):
    mesh = plsc.VectorSubcoreMesh(core_axis_name="c", subcore_axis_name="s",
                                  num_cores=NC, num_subcores=NS)
    out_type = [jax.ShapeDtypeStruct((NC, ACC_R, D), jnp.float32)]
    scratch = (
        [pltpu.VMEM((2, K), jnp.int32) for _ in range(NIB)]
        + [pltpu.VMEM((K, D), jnp.float32) for _ in range(NRB)]
        + [pltpu.VMEM((16, D), jnp.float32),
           pltpu.VMEM_SHARED((ACC_R, D), jnp.float32)]
        + [pltpu.SemaphoreType.DMA for _ in range(NIB)]
        + [pltpu.SemaphoreType.DMA for _ in range(NRB)]
        + [pltpu.SemaphoreType.DMA]
    )
    if with_deg:
        out_type.append(jax.ShapeDtypeStruct((NC, ACC_R), jnp.float32))
        scratch += [
            pltpu.VMEM((K,), jnp.float32),
            pltpu.VMEM((K,), jnp.float32),
            pltpu.VMEM_SHARED((ACC_R,), jnp.float32),
            pltpu.SemaphoreType.DMA,
        ]

    def body(feat_hbm, idx_hbm, *rest):
        no = 2 if with_deg else 1
        outs, rest = rest[:no], rest[no:]
        if with_deg:
            acc_out, deg_out = outs
            rest, (ones1, zde, deg_sh, semd) = rest[:-4], rest[-4:]
        else:
            acc_out, = outs
        idxb = rest[:NIB]
        rows = rest[NIB:NIB + NRB]
        zbuf, acc_sh = rest[NIB + NRB:NIB + NRB + 2]
        semi = rest[NIB + NRB + 2:NIB + NRB + 2 + NIB]
        semr = rest[NIB + NRB + 2 + NIB:NIB + NRB + 2 + NIB + NRB]
        sems = rest[NIB + NRB + 2 + NIB + NRB]
        cid = lax.axis_index("c")
        sid = lax.axis_index("s")
        wid = cid * NS + sid
        blk0 = wid * NBLK

        def fill(i, _):
            r = i // 8
            c = (i % 8) * 16
            zbuf[r, pl.ds(c, 16)] = jnp.zeros((16,), jnp.float32)
            if with_deg:
                o = (i % (K // 16)) * 16
                ones1[pl.ds(o, 16)] = jnp.ones((16,), jnp.float32)
                zde[pl.ds(o, 16)] = jnp.zeros((16,), jnp.float32)
            return 0
        lax.fori_loop(0, 16 * 8, fill, 0)

        for q in range(NIB):
            pltpu.async_copy(idx_hbm.at[blk0 + q], idxb[q], semi[q])
        for j in range(NRB):
            pltpu.make_async_copy(idx_hbm.at[blk0 + j], idxb[j],
                                  semi[j]).wait()
            pltpu.async_copy(feat_hbm.at[idxb[j].at[0]], rows[j], semr[j])

        def zero(b, _):
            pltpu.sync_copy(zbuf, acc_sh.at[pl.ds(sid * ZRPT + b * 16, 16)])
            return 0
        lax.fori_loop(0, ZRPT // 16, zero, 0)
        if with_deg:
            def zero2(b, _):
                pltpu.sync_copy(zde, deg_sh.at[pl.ds(sid * ZRPT + b * K, K)])
                return 0
            lax.fori_loop(0, ZRPT // K, zero2, 0)
        plsc.subcore_barrier()

        def octet(p, _):
            for j in range(NIB):
                b = NIB * p + j
                rj = j % NRB
                pltpu.make_async_copy(feat_hbm.at[idxb[j].at[0]],
                                      rows[rj], semr[rj]).wait()
                cp = pltpu.async_copy(rows[rj], acc_sh.at[idxb[j].at[1]],
                                      sems, add=True)
                if with_deg:
                    cpd = pltpu.async_copy(ones1, deg_sh.at[idxb[j].at[1]],
                                           semd, add=True)
                cp.wait()
                if with_deg:
                    cpd.wait()

                @pl.when(b + NIB < NBLK)
                def _():
                    pltpu.async_copy(idx_hbm.at[blk0 + b + NIB], idxb[j],
                                     semi[j])

                @pl.when(b + NRB < NBLK)
                def _():
                    qn = (j + NRB) % NIB
                    pltpu.make_async_copy(idx_hbm.at[blk0 + b + NRB],
                                          idxb[qn], semi[qn]).wait()
                    pltpu.async_copy(feat_hbm.at[idxb[qn].at[0]],
                                     rows[rj], semr[rj])
            return 0
        lax.fori_loop(0, NBLK // NIB, octet, 0)
        plsc.subcore_barrier()

        pltpu.sync_copy(acc_sh.at[pl.ds(sid * ZRPT, ZRPT)],
                        acc_out.at[cid, pl.ds(sid * ZRPT, ZRPT)])
        if with_deg:
            pltpu.sync_copy(deg_sh.at[pl.ds(sid * ZRPT, ZRPT)],
                            deg_out.at[cid, pl.ds(sid * ZRPT, ZRPT)])

    return pl.kernel(body, out_type=out_type, mesh=mesh,
                     scratch_types=scratch)


BR = 1024


def _dense_body(p0, p1, g, wt1, bt1, wt2, bt2, wp1, bp1, wp2, bp2, dout):
    rec = 1.0 / jnp.maximum(g[:, :], 1.0)
    a = (p0[:, :] + p1[:, :]) * rec
    hi = jax.lax.Precision.HIGHEST
    ht = jnp.maximum(
        jnp.dot(a, wt1[:, :], precision=hi,
                preferred_element_type=jnp.float32) + bt1[:, :], 0.0)
    hp = jnp.maximum(
        jnp.dot(a, wp1[:, :], precision=hi,
                preferred_element_type=jnp.float32) + bp1[:, :], 0.0)
    dout[:, :] = (jnp.dot(ht, wt2[:, :], precision=hi,
                          preferred_element_type=jnp.float32)
                  - jnp.dot(hp, wp2[:, :], precision=hi,
                            preferred_element_type=jnp.float32))


def _loss_body(p0, p1, g, db, out):
    i = pl.program_id(0)
    rec = 1.0 / jnp.maximum(g[:, :], 1.0)
    t = (p0[:, :] + p1[:, :]) * rec + db[:, :]
    row = i * BR + jax.lax.broadcasted_iota(jnp.int32, (BR, D), 0)
    t = jnp.where(row < N, t, 0.0)
    s = jnp.reshape(jnp.sum(t * t) * (1.0 / (N * D)), (1, 1))

    @pl.when(i == 0)
    def _():
        out[:, :] = s

    @pl.when(i > 0)
    def _():
        out[:, :] += s


def _row_spec(w):
    return pl.BlockSpec((BR, w), lambda i: (i, 0))


def _full_spec(r, c):
    return pl.BlockSpec((r, c), lambda i: (0, 0))


@jax.jit
def kernel(x, edge_index, Wt1, bt1, Wt2, bt2, Wp1, bp1, Wp2, bp2):
    src = edge_index[0].astype(jnp.int32)
    dst = edge_index[1].astype(jnp.int32)
    pad = EP - E
    src_p = jnp.concatenate([src, jnp.zeros((pad,), jnp.int32)])
    dst_p = jnp.concatenate([dst, jnp.full((pad,), N, jnp.int32)])
    idx3 = jnp.stack([src_p.reshape(EP // K, K),
                      dst_p.reshape(EP // K, K)], axis=1)

    acc1, degp = _seg_kernel(with_deg=True)(x, idx3)
    deg = (degp[0] + degp[1]).reshape(ACC_R, 1)

    d = pl.pallas_call(
        _dense_body,
        grid=(ACC_R // BR,),
        in_specs=[_row_spec(D), _row_spec(D), _row_spec(1),
                  _full_spec(D, D), _full_spec(1, D), _full_spec(D, D),
                  _full_spec(1, D), _full_spec(D, D), _full_spec(1, D),
                  _full_spec(D, D), _full_spec(1, D)],
        out_specs=_row_spec(D),
        out_shape=jax.ShapeDtypeStruct((ACC_R, D), jnp.float32),
    )(acc1[0], acc1[1], deg,
      Wt1, bt1.reshape(1, D), Wt2, bt2.reshape(1, D),
      Wp1, bp1.reshape(1, D), Wp2, bp2.reshape(1, D))

    acc2, = _seg_kernel(with_deg=False)(d, idx3)

    db = (bt2 - bp2).reshape(1, D)
    loss = pl.pallas_call(
        _loss_body,
        grid=(ACC_R // BR,),
        in_specs=[_row_spec(D), _row_spec(D), _row_spec(1), _full_spec(1, D)],
        out_specs=_full_spec(1, 1),
        out_shape=jax.ShapeDtypeStruct((1, 1), jnp.float32),
    )(acc2[0], acc2[1], deg, db)

    return loss[0, 0]


# speedup vs baseline: 1.0557x; 1.0009x over previous
import jax
import jax.numpy as jnp
from jax import lax
from jax.experimental import pallas as pl
from jax.experimental.pallas import tpu as pltpu
from jax.experimental.pallas import tpu_sc as plsc

N = 10000
E = 320000
D = 128
NC, NS = 2, 16
NW = NC * NS
EP = 327680
EPT = EP // NW
K = 128
NBLK = EPT // K
NRB = 2
NIB = 4
ACC_R = 10240
ZRPT = ACC_R // NS


def _seg_kernel(with_deg: bool):
    mesh = plsc.VectorSubcoreMesh(core_axis_name="c", subcore_axis_name="s",
                                  num_cores=NC, num_subcores=NS)
    out_type = [jax.ShapeDtypeStruct((NC, ACC_R, D), jnp.float32)]
    scratch = (
        [pltpu.VMEM((2, K), jnp.int32) for _ in range(NIB)]
        + [pltpu.VMEM((K, D), jnp.float32) for _ in range(NRB)]
        + [pltpu.VMEM((16, D), jnp.float32),
           pltpu.VMEM_SHARED((ACC_R, D), jnp.float32)]
        + [pltpu.SemaphoreType.DMA for _ in range(NIB)]
        + [pltpu.SemaphoreType.DMA for _ in range(NRB)]
        + [pltpu.SemaphoreType.DMA]
    )
    if with_deg:
        out_type.append(jax.ShapeDtypeStruct((NC, ACC_R), jnp.float32))
        scratch += [
            pltpu.VMEM((K,), jnp.float32),
            pltpu.VMEM((K,), jnp.float32),
            pltpu.VMEM_SHARED((ACC_R,), jnp.float32),
            pltpu.SemaphoreType.DMA,
        ]

    def body(feat_hbm, idx_hbm, *rest):
        no = 2 if with_deg else 1
        outs, rest = rest[:no], rest[no:]
        if with_deg:
            acc_out, deg_out = outs
            rest, (ones1, zde, deg_sh, semd) = rest[:-4], rest[-4:]
        else:
            acc_out, = outs
        idxb = rest[:NIB]
        rows = rest[NIB:NIB + NRB]
        zbuf, acc_sh = rest[NIB + NRB:NIB + NRB + 2]
        semi = rest[NIB + NRB + 2:NIB + NRB + 2 + NIB]
        semr = rest[NIB + NRB + 2 + NIB:NIB + NRB + 2 + NIB + NRB]
        sems = rest[NIB + NRB + 2 + NIB + NRB]
        cid = lax.axis_index("c")
        sid = lax.axis_index("s")
        wid = cid * NS + sid
        blk0 = wid * NBLK

        def fill(i, _):
            r = i // 8
            c = (i % 8) * 16
            zbuf[r, pl.ds(c, 16)] = jnp.zeros((16,), jnp.float32)
            if with_deg:
                o = (i % (K // 16)) * 16
                ones1[pl.ds(o, 16)] = jnp.ones((16,), jnp.float32)
                zde[pl.ds(o, 16)] = jnp.zeros((16,), jnp.float32)
            return 0
        lax.fori_loop(0, 16 * 8, fill, 0)

        for q in range(NIB):
            pltpu.async_copy(idx_hbm.at[blk0 + q], idxb[q], semi[q])
        for j in range(NRB):
            pltpu.make_async_copy(idx_hbm.at[blk0 + j], idxb[j],
                                  semi[j]).wait()
            pltpu.async_copy(feat_hbm.at[idxb[j].at[0]], rows[j], semr[j])

        def zero(b, _):
            pltpu.sync_copy(zbuf, acc_sh.at[pl.ds(sid * ZRPT + b * 16, 16)])
            return 0
        lax.fori_loop(0, ZRPT // 16, zero, 0)
        if with_deg:
            def zero2(b, _):
                pltpu.sync_copy(zde, deg_sh.at[pl.ds(sid * ZRPT + b * K, K)])
                return 0
            lax.fori_loop(0, ZRPT // K, zero2, 0)
        plsc.subcore_barrier()

        def octet(p, _):
            for j in range(NIB):
                b = NIB * p + j
                rj = j % NRB
                pltpu.make_async_copy(feat_hbm.at[idxb[j].at[0]],
                                      rows[rj], semr[rj]).wait()
                cp = pltpu.async_copy(rows[rj], acc_sh.at[idxb[j].at[1]],
                                      sems, add=True)
                if with_deg:
                    cpd = pltpu.async_copy(ones1, deg_sh.at[idxb[j].at[1]],
                                           semd, add=True)
                cp.wait()
                if with_deg:
                    cpd.wait()

                @pl.when(b + NIB < NBLK)
                def _():
                    pltpu.async_copy(idx_hbm.at[blk0 + b + NIB], idxb[j],
                                     semi[j])

                @pl.when(b + NRB < NBLK)
                def _():
                    qn = (j + NRB) % NIB
                    pltpu.make_async_copy(idx_hbm.at[blk0 + b + NRB],
                                          idxb[qn], semi[qn]).wait()
                    pltpu.async_copy(feat_hbm.at[idxb[qn].at[0]],
                                     rows[rj], semr[rj])
            return 0
        lax.fori_loop(0, NBLK // NIB, octet, 0)
        plsc.subcore_barrier()

        pltpu.sync_copy(acc_sh.at[pl.ds(sid * ZRPT, ZRPT)],
                        acc_out.at[cid, pl.ds(sid * ZRPT, ZRPT)])
        if with_deg:
            pltpu.sync_copy(deg_sh.at[pl.ds(sid * ZRPT, ZRPT)],
                            deg_out.at[cid, pl.ds(sid * ZRPT, ZRPT)])

    return pl.kernel(body, out_type=out_type, mesh=mesh,
                     scratch_types=scratch)


BR = 1024


def _dense_body(p0, p1, g, wt1, bt1, wt2, bt2, wp1, bp1, wp2, bp2, dout):
    rec = 1.0 / jnp.maximum(g[:, :], 1.0)
    a = (p0[:, :] + p1[:, :]) * rec
    hi = jax.lax.Precision.HIGHEST
    ht = jnp.maximum(
        jnp.dot(a, wt1[:, :], precision=hi,
                preferred_element_type=jnp.float32) + bt1[:, :], 0.0)
    hp = jnp.maximum(
        jnp.dot(a, wp1[:, :], precision=hi,
                preferred_element_type=jnp.float32) + bp1[:, :], 0.0)
    dout[:, :] = (jnp.dot(ht, wt2[:, :], precision=hi,
                          preferred_element_type=jnp.float32)
                  - jnp.dot(hp, wp2[:, :], precision=hi,
                            preferred_element_type=jnp.float32))


def _loss_body(p0, p1, g, db, out):
    i = pl.program_id(0)
    rec = 1.0 / jnp.maximum(g[:, :], 1.0)
    t = (p0[:, :] + p1[:, :]) * rec + db[:, :]
    row = i * BR + jax.lax.broadcasted_iota(jnp.int32, (BR, D), 0)
    t = jnp.where(row < N, t, 0.0)
    s = jnp.reshape(jnp.sum(t * t) * (1.0 / (N * D)), (1, 1))

    @pl.when(i == 0)
    def _():
        out[:, :] = s

    @pl.when(i > 0)
    def _():
        out[:, :] += s


def _row_spec(w):
    return pl.BlockSpec((BR, w), lambda i: (i, 0))


def _full_spec(r, c):
    return pl.BlockSpec((r, c), lambda i: (0, 0))


@jax.jit
def kernel(x, edge_index, Wt1, bt1, Wt2, bt2, Wp1, bp1, Wp2, bp2):
    src = edge_index[0].astype(jnp.int32)
    dst = edge_index[1].astype(jnp.int32)
    pad = EP - E
    src_p = jnp.concatenate([src, jnp.zeros((pad,), jnp.int32)])
    trash = N + (jnp.arange(pad, dtype=jnp.int32) % (ACC_R - N))
    dst_p = jnp.concatenate([dst, trash])
    idx3 = jnp.stack([src_p.reshape(EP // K, K),
                      dst_p.reshape(EP // K, K)], axis=1)

    acc1, degp = _seg_kernel(with_deg=True)(x, idx3)
    deg = (degp[0] + degp[1]).reshape(ACC_R, 1)

    d = pl.pallas_call(
        _dense_body,
        grid=(ACC_R // BR,),
        in_specs=[_row_spec(D), _row_spec(D), _row_spec(1),
                  _full_spec(D, D), _full_spec(1, D), _full_spec(D, D),
                  _full_spec(1, D), _full_spec(D, D), _full_spec(1, D),
                  _full_spec(D, D), _full_spec(1, D)],
        out_specs=_row_spec(D),
        out_shape=jax.ShapeDtypeStruct((ACC_R, D), jnp.float32),
    )(acc1[0], acc1[1], deg,
      Wt1, bt1.reshape(1, D), Wt2, bt2.reshape(1, D),
      Wp1, bp1.reshape(1, D), Wp2, bp2.reshape(1, D))

    acc2, = _seg_kernel(with_deg=False)(d, idx3)

    db = (bt2 - bp2).reshape(1, D)
    loss = pl.pallas_call(
        _loss_body,
        grid=(ACC_R // BR,),
        in_specs=[_row_spec(D), _row_spec(D), _row_spec(1), _full_spec(1, D)],
        out_specs=_full_spec(1, 1),
        out_shape=jax.ShapeDtypeStruct((1, 1), jnp.float32),
    )(acc2[0], acc2[1], deg, db)

    return loss[0, 0]


# speedup vs baseline: 1.1124x; 1.0538x over previous
import jax
import jax.numpy as jnp
from jax import lax
from jax.experimental import pallas as pl
from jax.experimental.pallas import tpu as pltpu
from jax.experimental.pallas import tpu_sc as plsc

N = 10000
E = 320000
D = 128
NC, NS = 2, 16
NW = NC * NS
EP = 327680
EPT = EP // NW
K = 128
NBLK = EPT // K
NRB = 2
NIB = 4
B0 = 108
B1 = 2 * NBLK - B0
ACC_R = 10240
ZRPT = ACC_R // NS


def _seg_kernel(with_deg: bool):
    mesh = plsc.VectorSubcoreMesh(core_axis_name="c", subcore_axis_name="s",
                                  num_cores=NC, num_subcores=NS)
    out_type = [jax.ShapeDtypeStruct((NC, ACC_R, D), jnp.float32)]
    scratch = (
        [pltpu.VMEM((2, K), jnp.int32) for _ in range(NIB)]
        + [pltpu.VMEM((K, D), jnp.float32) for _ in range(NRB)]
        + [pltpu.VMEM((16, D), jnp.float32),
           pltpu.VMEM_SHARED((ACC_R, D), jnp.float32)]
        + [pltpu.SemaphoreType.DMA for _ in range(NIB)]
        + [pltpu.SemaphoreType.DMA for _ in range(NRB)]
        + [pltpu.SemaphoreType.DMA]
    )
    if with_deg:
        out_type.append(jax.ShapeDtypeStruct((NC, ACC_R), jnp.float32))
        scratch += [
            pltpu.VMEM((K,), jnp.float32),
            pltpu.VMEM((K,), jnp.float32),
            pltpu.VMEM_SHARED((ACC_R,), jnp.float32),
            pltpu.SemaphoreType.DMA,
        ]

    def body(feat_hbm, idx_hbm, *rest):
        no = 2 if with_deg else 1
        outs, rest = rest[:no], rest[no:]
        if with_deg:
            acc_out, deg_out = outs
            rest, (ones1, zde, deg_sh, semd) = rest[:-4], rest[-4:]
        else:
            acc_out, = outs
        idxb = rest[:NIB]
        rows = rest[NIB:NIB + NRB]
        zbuf, acc_sh = rest[NIB + NRB:NIB + NRB + 2]
        semi = rest[NIB + NRB + 2:NIB + NRB + 2 + NIB]
        semr = rest[NIB + NRB + 2 + NIB:NIB + NRB + 2 + NIB + NRB]
        sems = rest[NIB + NRB + 2 + NIB + NRB]
        cid = lax.axis_index("c")
        sid = lax.axis_index("s")
        nb = jnp.where(cid == 0, B0, B1)
        blk0 = cid * NS * B0 + sid * nb

        def fill(i, _):
            r = i // 8
            c = (i % 8) * 16
            zbuf[r, pl.ds(c, 16)] = jnp.zeros((16,), jnp.float32)
            if with_deg:
                o = (i % (K // 16)) * 16
                ones1[pl.ds(o, 16)] = jnp.ones((16,), jnp.float32)
                zde[pl.ds(o, 16)] = jnp.zeros((16,), jnp.float32)
            return 0
        lax.fori_loop(0, 16 * 8, fill, 0)

        for q in range(NIB):
            pltpu.async_copy(idx_hbm.at[blk0 + q], idxb[q], semi[q])
        for j in range(NRB):
            pltpu.make_async_copy(idx_hbm.at[blk0 + j], idxb[j],
                                  semi[j]).wait()
            pltpu.async_copy(feat_hbm.at[idxb[j].at[0]], rows[j], semr[j])

        def zero(b, _):
            pltpu.sync_copy(zbuf, acc_sh.at[pl.ds(sid * ZRPT + b * 16, 16)])
            return 0
        lax.fori_loop(0, ZRPT // 16, zero, 0)
        if with_deg:
            def zero2(b, _):
                pltpu.sync_copy(zde, deg_sh.at[pl.ds(sid * ZRPT + b * K, K)])
                return 0
            lax.fori_loop(0, ZRPT // K, zero2, 0)
        plsc.subcore_barrier()

        def octet(p, _):
            for j in range(NIB):
                b = NIB * p + j
                rj = j % NRB
                pltpu.make_async_copy(feat_hbm.at[idxb[j].at[0]],
                                      rows[rj], semr[rj]).wait()
                cp = pltpu.async_copy(rows[rj], acc_sh.at[idxb[j].at[1]],
                                      sems, add=True)
                if with_deg:
                    cpd = pltpu.async_copy(ones1, deg_sh.at[idxb[j].at[1]],
                                           semd, add=True)
                cp.wait()
                if with_deg:
                    cpd.wait()

                @pl.when(b + NIB < nb)
                def _():
                    pltpu.async_copy(idx_hbm.at[blk0 + b + NIB], idxb[j],
                                     semi[j])

                @pl.when(b + NRB < nb)
                def _():
                    qn = (j + NRB) % NIB
                    pltpu.make_async_copy(idx_hbm.at[blk0 + b + NRB],
                                          idxb[qn], semi[qn]).wait()
                    pltpu.async_copy(feat_hbm.at[idxb[qn].at[0]],
                                     rows[rj], semr[rj])
            return 0
        lax.fori_loop(0, nb // NIB, octet, 0)
        plsc.subcore_barrier()

        pltpu.sync_copy(acc_sh.at[pl.ds(sid * ZRPT, ZRPT)],
                        acc_out.at[cid, pl.ds(sid * ZRPT, ZRPT)])
        if with_deg:
            pltpu.sync_copy(deg_sh.at[pl.ds(sid * ZRPT, ZRPT)],
                            deg_out.at[cid, pl.ds(sid * ZRPT, ZRPT)])

    return pl.kernel(body, out_type=out_type, mesh=mesh,
                     scratch_types=scratch)


BR = 1024


def _dense_body(p0, p1, g, wt1, bt1, wt2, bt2, wp1, bp1, wp2, bp2, dout):
    rec = 1.0 / jnp.maximum(g[:, :], 1.0)
    a = (p0[:, :] + p1[:, :]) * rec
    hi = jax.lax.Precision.HIGHEST
    ht = jnp.maximum(
        jnp.dot(a, wt1[:, :], precision=hi,
                preferred_element_type=jnp.float32) + bt1[:, :], 0.0)
    hp = jnp.maximum(
        jnp.dot(a, wp1[:, :], precision=hi,
                preferred_element_type=jnp.float32) + bp1[:, :], 0.0)
    dout[:, :] = (jnp.dot(ht, wt2[:, :], precision=hi,
                          preferred_element_type=jnp.float32)
                  - jnp.dot(hp, wp2[:, :], precision=hi,
                            preferred_element_type=jnp.float32))


def _loss_body(p0, p1, g, db, out):
    i = pl.program_id(0)
    rec = 1.0 / jnp.maximum(g[:, :], 1.0)
    t = (p0[:, :] + p1[:, :]) * rec + db[:, :]
    row = i * BR + jax.lax.broadcasted_iota(jnp.int32, (BR, D), 0)
    t = jnp.where(row < N, t, 0.0)
    s = jnp.reshape(jnp.sum(t * t) * (1.0 / (N * D)), (1, 1))

    @pl.when(i == 0)
    def _():
        out[:, :] = s

    @pl.when(i > 0)
    def _():
        out[:, :] += s


def _row_spec(w):
    return pl.BlockSpec((BR, w), lambda i: (i, 0))


def _full_spec(r, c):
    return pl.BlockSpec((r, c), lambda i: (0, 0))


@jax.jit
def kernel(x, edge_index, Wt1, bt1, Wt2, bt2, Wp1, bp1, Wp2, bp2):
    src = edge_index[0].astype(jnp.int32)
    dst = edge_index[1].astype(jnp.int32)
    pad = EP - E
    src_p = jnp.concatenate([src, jnp.zeros((pad,), jnp.int32)])
    trash = N + (jnp.arange(pad, dtype=jnp.int32) % (ACC_R - N))
    dst_p = jnp.concatenate([dst, trash])
    idx3 = jnp.stack([src_p.reshape(EP // K, K),
                      dst_p.reshape(EP // K, K)], axis=1)

    acc1, degp = _seg_kernel(with_deg=True)(x, idx3)
    deg = (degp[0] + degp[1]).reshape(ACC_R, 1)

    d = pl.pallas_call(
        _dense_body,
        grid=(ACC_R // BR,),
        in_specs=[_row_spec(D), _row_spec(D), _row_spec(1),
                  _full_spec(D, D), _full_spec(1, D), _full_spec(D, D),
                  _full_spec(1, D), _full_spec(D, D), _full_spec(1, D),
                  _full_spec(D, D), _full_spec(1, D)],
        out_specs=_row_spec(D),
        out_shape=jax.ShapeDtypeStruct((ACC_R, D), jnp.float32),
    )(acc1[0], acc1[1], deg,
      Wt1, bt1.reshape(1, D), Wt2, bt2.reshape(1, D),
      Wp1, bp1.reshape(1, D), Wp2, bp2.reshape(1, D))

    acc2, = _seg_kernel(with_deg=False)(d, idx3)

    db = (bt2 - bp2).reshape(1, D)
    loss = pl.pallas_call(
        _loss_body,
        grid=(ACC_R // BR,),
        in_specs=[_row_spec(D), _row_spec(D), _row_spec(1), _full_spec(1, D)],
        out_specs=_full_spec(1, 1),
        out_shape=jax.ShapeDtypeStruct((1, 1), jnp.float32),
    )(acc2[0], acc2[1], deg, db)

    return loss[0, 0]


# speedup vs baseline: 1.1231x; 1.0096x over previous
import jax
import jax.numpy as jnp
from jax import lax
from jax.experimental import pallas as pl
from jax.experimental.pallas import tpu as pltpu
from jax.experimental.pallas import tpu_sc as plsc

N = 10000
E = 320000
D = 128
NC, NS = 2, 16
NW = NC * NS
EP = 327680
EPT = EP // NW
K = 128
NBLK = EPT // K
NRB = 2
NIB = 4
B0 = 120
B1 = 2 * NBLK - B0
ACC_R = 10240
ZRPT = ACC_R // NS


def _seg_kernel(with_deg: bool):
    mesh = plsc.VectorSubcoreMesh(core_axis_name="c", subcore_axis_name="s",
                                  num_cores=NC, num_subcores=NS)
    out_type = [jax.ShapeDtypeStruct((NC, ACC_R, D), jnp.float32)]
    scratch = (
        [pltpu.VMEM((2, K), jnp.int32) for _ in range(NIB)]
        + [pltpu.VMEM((K, D), jnp.float32) for _ in range(NRB)]
        + [pltpu.VMEM((16, D), jnp.float32),
           pltpu.VMEM_SHARED((ACC_R, D), jnp.float32)]
        + [pltpu.SemaphoreType.DMA for _ in range(NIB)]
        + [pltpu.SemaphoreType.DMA for _ in range(NRB)]
        + [pltpu.SemaphoreType.DMA]
    )
    if with_deg:
        out_type.append(jax.ShapeDtypeStruct((NC, ACC_R), jnp.float32))
        scratch += [
            pltpu.VMEM((K,), jnp.float32),
            pltpu.VMEM((K,), jnp.float32),
            pltpu.VMEM_SHARED((ACC_R,), jnp.float32),
            pltpu.SemaphoreType.DMA,
        ]

    def body(feat_hbm, idx_hbm, *rest):
        no = 2 if with_deg else 1
        outs, rest = rest[:no], rest[no:]
        if with_deg:
            acc_out, deg_out = outs
            rest, (ones1, zde, deg_sh, semd) = rest[:-4], rest[-4:]
        else:
            acc_out, = outs
        idxb = rest[:NIB]
        rows = rest[NIB:NIB + NRB]
        zbuf, acc_sh = rest[NIB + NRB:NIB + NRB + 2]
        semi = rest[NIB + NRB + 2:NIB + NRB + 2 + NIB]
        semr = rest[NIB + NRB + 2 + NIB:NIB + NRB + 2 + NIB + NRB]
        sems = rest[NIB + NRB + 2 + NIB + NRB]
        cid = lax.axis_index("c")
        sid = lax.axis_index("s")
        nb = jnp.where(cid == 0, B0, B1)
        blk0 = cid * NS * B0 + sid * nb

        def fill(i, _):
            r = i // 8
            c = (i % 8) * 16
            zbuf[r, pl.ds(c, 16)] = jnp.zeros((16,), jnp.float32)
            if with_deg:
                o = (i % (K // 16)) * 16
                ones1[pl.ds(o, 16)] = jnp.ones((16,), jnp.float32)
                zde[pl.ds(o, 16)] = jnp.zeros((16,), jnp.float32)
            return 0
        lax.fori_loop(0, 16 * 8, fill, 0)

        for q in range(NIB):
            pltpu.async_copy(idx_hbm.at[blk0 + q], idxb[q], semi[q])
        for j in range(NRB):
            pltpu.make_async_copy(idx_hbm.at[blk0 + j], idxb[j],
                                  semi[j]).wait()
            pltpu.async_copy(feat_hbm.at[idxb[j].at[0]], rows[j], semr[j])

        def zero(b, _):
            pltpu.sync_copy(zbuf, acc_sh.at[pl.ds(sid * ZRPT + b * 16, 16)])
            return 0
        lax.fori_loop(0, ZRPT // 16, zero, 0)
        if with_deg:
            def zero2(b, _):
                pltpu.sync_copy(zde, deg_sh.at[pl.ds(sid * ZRPT + b * K, K)])
                return 0
            lax.fori_loop(0, ZRPT // K, zero2, 0)
        plsc.subcore_barrier()

        def octet(p, _):
            for j in range(NIB):
                b = NIB * p + j
                rj = j % NRB
                pltpu.make_async_copy(feat_hbm.at[idxb[j].at[0]],
                                      rows[rj], semr[rj]).wait()
                cp = pltpu.async_copy(rows[rj], acc_sh.at[idxb[j].at[1]],
                                      sems, add=True)
                if with_deg:
                    cpd = pltpu.async_copy(ones1, deg_sh.at[idxb[j].at[1]],
                                           semd, add=True)
                cp.wait()
                if with_deg:
                    cpd.wait()

                @pl.when(b + NIB < nb)
                def _():
                    pltpu.async_copy(idx_hbm.at[blk0 + b + NIB], idxb[j],
                                     semi[j])

                @pl.when(b + NRB < nb)
                def _():
                    qn = (j + NRB) % NIB
                    pltpu.make_async_copy(idx_hbm.at[blk0 + b + NRB],
                                          idxb[qn], semi[qn]).wait()
                    pltpu.async_copy(feat_hbm.at[idxb[qn].at[0]],
                                     rows[rj], semr[rj])
            return 0
        lax.fori_loop(0, nb // NIB, octet, 0)
        plsc.subcore_barrier()

        pltpu.sync_copy(acc_sh.at[pl.ds(sid * ZRPT, ZRPT)],
                        acc_out.at[cid, pl.ds(sid * ZRPT, ZRPT)])
        if with_deg:
            pltpu.sync_copy(deg_sh.at[pl.ds(sid * ZRPT, ZRPT)],
                            deg_out.at[cid, pl.ds(sid * ZRPT, ZRPT)])

    return pl.kernel(body, out_type=out_type, mesh=mesh,
                     scratch_types=scratch)


BR = 1024


def _dense_body(p0, p1, g, wt1, bt1, wt2, bt2, wp1, bp1, wp2, bp2, dout):
    rec = 1.0 / jnp.maximum(g[:, :], 1.0)
    a = (p0[:, :] + p1[:, :]) * rec
    hi = jax.lax.Precision.HIGHEST
    ht = jnp.maximum(
        jnp.dot(a, wt1[:, :], precision=hi,
                preferred_element_type=jnp.float32) + bt1[:, :], 0.0)
    hp = jnp.maximum(
        jnp.dot(a, wp1[:, :], precision=hi,
                preferred_element_type=jnp.float32) + bp1[:, :], 0.0)
    dout[:, :] = (jnp.dot(ht, wt2[:, :], precision=hi,
                          preferred_element_type=jnp.float32)
                  - jnp.dot(hp, wp2[:, :], precision=hi,
                            preferred_element_type=jnp.float32))


def _loss_body(p0, p1, g, db, out):
    i = pl.program_id(0)
    rec = 1.0 / jnp.maximum(g[:, :], 1.0)
    t = (p0[:, :] + p1[:, :]) * rec + db[:, :]
    row = i * BR + jax.lax.broadcasted_iota(jnp.int32, (BR, D), 0)
    t = jnp.where(row < N, t, 0.0)
    s = jnp.reshape(jnp.sum(t * t) * (1.0 / (N * D)), (1, 1))

    @pl.when(i == 0)
    def _():
        out[:, :] = s

    @pl.when(i > 0)
    def _():
        out[:, :] += s


def _row_spec(w):
    return pl.BlockSpec((BR, w), lambda i: (i, 0))


def _full_spec(r, c):
    return pl.BlockSpec((r, c), lambda i: (0, 0))


@jax.jit
def kernel(x, edge_index, Wt1, bt1, Wt2, bt2, Wp1, bp1, Wp2, bp2):
    src = edge_index[0].astype(jnp.int32)
    dst = edge_index[1].astype(jnp.int32)
    pad = EP - E
    src_p = jnp.concatenate([src, jnp.zeros((pad,), jnp.int32)])
    trash = N + (jnp.arange(pad, dtype=jnp.int32) % (ACC_R - N))
    dst_p = jnp.concatenate([dst, trash])
    idx3 = jnp.stack([src_p.reshape(EP // K, K),
                      dst_p.reshape(EP // K, K)], axis=1)

    acc1, degp = _seg_kernel(with_deg=True)(x, idx3)
    deg = (degp[0] + degp[1]).reshape(ACC_R, 1)

    d = pl.pallas_call(
        _dense_body,
        grid=(ACC_R // BR,),
        in_specs=[_row_spec(D), _row_spec(D), _row_spec(1),
                  _full_spec(D, D), _full_spec(1, D), _full_spec(D, D),
                  _full_spec(1, D), _full_spec(D, D), _full_spec(1, D),
                  _full_spec(D, D), _full_spec(1, D)],
        out_specs=_row_spec(D),
        out_shape=jax.ShapeDtypeStruct((ACC_R, D), jnp.float32),
    )(acc1[0], acc1[1], deg,
      Wt1, bt1.reshape(1, D), Wt2, bt2.reshape(1, D),
      Wp1, bp1.reshape(1, D), Wp2, bp2.reshape(1, D))

    acc2, = _seg_kernel(with_deg=False)(d, idx3)

    db = (bt2 - bp2).reshape(1, D)
    loss = pl.pallas_call(
        _loss_body,
        grid=(ACC_R // BR,),
        in_specs=[_row_spec(D), _row_spec(D), _row_spec(1), _full_spec(1, D)],
        out_specs=_full_spec(1, 1),
        out_shape=jax.ShapeDtypeStruct((1, 1), jnp.float32),
    )(acc2[0], acc2[1], deg, db)

    return loss[0, 0]
